# baseline (speedup 1.0000x reference)
import functools

import jax
import jax.numpy as jnp
from jax import lax
from jax.experimental import pallas as pl
from jax.experimental.pallas import tpu as pltpu
from jax.experimental.pallas import tpu_sc as plsc

D = 256
L = 16
NSC = 2
NTEC = 16
ROWS_PER_TEC = 320
OUT_CHUNK = 40
CHUNK = 2000
CBUF = 2176
FILT_UNROLL = 5
BG = 128


def _mm_body(x_ref, w_ref, o_ref):
    o_ref[...] = jnp.dot(x_ref[...], w_ref[...],
                         preferred_element_type=jnp.float32)


def _matmul(x, W):
    n, d = x.shape
    blk = 1000
    return pl.pallas_call(
        _mm_body,
        grid=(n // blk,),
        in_specs=[
            pl.BlockSpec((blk, d), lambda i: (i, 0)),
            pl.BlockSpec((d, W.shape[1]), lambda i: (0, 0)),
        ],
        out_specs=pl.BlockSpec((blk, W.shape[1]), lambda i: (i, 0)),
        out_shape=jax.ShapeDtypeStruct((n, W.shape[1]), jnp.float32),
    )(x, W)


def _make_sc_agg(n_nodes, n_edges):
    n_chunks = n_edges // CHUNK
    filt_iters = CHUNK // L
    out_copies = ROWS_PER_TEC // OUT_CHUNK
    mesh = plsc.VectorSubcoreMesh(core_axis_name="c", subcore_axis_name="s")

    @functools.partial(
        pl.kernel,
        out_type=jax.ShapeDtypeStruct((n_nodes, D), jnp.float32),
        mesh=mesh,
        compiler_params=pltpu.CompilerParams(needs_layout_passes=False),
        scratch_types=[
            pltpu.VMEM((ROWS_PER_TEC, D), jnp.float32),
            pltpu.VMEM((CHUNK,), jnp.int32),
            pltpu.VMEM((CHUNK,), jnp.int32),
            pltpu.VMEM((CHUNK,), jnp.float32),
            pltpu.VMEM((CBUF,), jnp.int32),
            pltpu.VMEM((CBUF,), jnp.int32),
            pltpu.VMEM((CBUF,), jnp.float32),
            pltpu.VMEM((BG, D), jnp.float32),
            pltpu.VMEM((BG,), jnp.int32),
            pltpu.VMEM((D,), jnp.float32),
            pltpu.VMEM((L,), jnp.int32),
            pltpu.SemaphoreType.DMA,
            pltpu.SemaphoreType.DMA,
            pltpu.SemaphoreType.DMA,
        ],
    )
    def sc_agg(h_hbm, dst_hbm, src_hbm, w_hbm, b_hbm, out_hbm,
               acc, dstb, srcb, wb, cloc, csrc, cw, rowbig, idxbuf,
               biasv, ctmp, sem0, sem1, esem):
        c = lax.axis_index("c")
        s = lax.axis_index("s")
        wid = c * NTEC + s
        base = wid * ROWS_PER_TEC
        zf = jnp.zeros((L,), jnp.float32)
        zi = jnp.zeros((L,), jnp.int32)
        iota = lax.iota(jnp.int32, L)

        def _zrow(r, _):
            for k in range(D // L):
                acc[r, pl.ds(k * L, L)] = zf
            return 0
        lax.fori_loop(0, ROWS_PER_TEC, _zrow, 0)
        pltpu.sync_copy(b_hbm, biasv)

        last15 = jnp.full((L,), L - 1, jnp.int32)

        def _proc16(locv, wv, row0, buf):
            for r in range(L):
                loc_s = locv[r]
                wrv = jnp.full((L,), wv[r])
                row = row0 + r

                @plsc.parallel_loop(0, D // L, step=1, unroll=16)
                def _k(k):
                    v = buf[row, pl.ds(k * L, L)] * wrv
                    plsc.addupdate(acc.at[loc_s, pl.ds(k * L, L)], v)

        def _chunk(ch, cnt_in):
            off0 = ch * CHUNK
            pltpu.async_copy(dst_hbm.at[pl.ds(off0, CHUNK)], dstb, esem)
            pltpu.async_copy(src_hbm.at[pl.ds(off0, CHUNK)], srcb, esem)
            copy_w = pltpu.async_copy(w_hbm.at[pl.ds(off0, CHUNK)], wb,
                                      esem)
            pltpu.make_async_copy(dst_hbm.at[pl.ds(off0, CHUNK)], dstb,
                                  esem).wait()
            pltpu.make_async_copy(src_hbm.at[pl.ds(off0, CHUNK)], srcb,
                                  esem).wait()
            copy_w.wait()

            @plsc.parallel_loop(0, filt_iters, step=1, unroll=FILT_UNROLL,
                                carry=cnt_in)
            def _filt(j, off):
                d = dstb[pl.ds(j * L, L)]
                sv = srcb[pl.ds(j * L, L)]
                wv = wb[pl.ds(j * L, L)]
                loc = d - base
                m = (loc >= 0) & (loc < ROWS_PER_TEC)
                mi = jnp.where(m, 1, 0)
                cum = plsc.cumsum(mi)
                pos = off + cum - 1
                plsc.store_scatter(cloc, [pos], loc, mask=m)
                plsc.store_scatter(csrc, [pos], sv, mask=m)
                plsc.store_scatter(cw, [pos], wv, mask=m)
                return off + cum[L - 1]

            cnt = _filt
            ndrain = cnt // BG

            def _dg(dg, _):
                b0 = dg * BG
                for k in range(BG // L):
                    idxbuf[pl.ds(k * L, L)] = csrc[pl.ds(b0 + k * L, L)]
                pltpu.async_copy(h_hbm.at[idxbuf], rowbig, sem0).wait()

                def _pblock(i, _):
                    bk16 = b0 + i * L
                    _proc16(cloc[pl.ds(bk16, L)], cw[pl.ds(bk16, L)],
                            i * L, rowbig)
                    return 0

                lax.fori_loop(0, BG // L, _pblock, 0)
                return 0

            lax.fori_loop(0, ndrain, _dg, 0)

            rem0 = ndrain * BG

            @pl.when(ndrain > 0)
            def _():
                for k in range(BG // L):
                    src_sl = pl.ds(rem0 + k * L, L)
                    dst_sl = pl.ds(k * L, L)
                    cloc[dst_sl] = cloc[src_sl]
                    csrc[dst_sl] = csrc[src_sl]
                    cw[dst_sl] = cw[src_sl]

            return cnt - rem0

        rem = lax.fori_loop(0, n_chunks, _chunk, jnp.int32(0))

        cloc[pl.ds(rem, L)] = zi
        csrc[pl.ds(rem, L)] = zi
        cw[pl.ds(rem, L)] = zf
        nb_f = (rem + (L - 1)) // L

        def _fblock(bk, _):
            idxv = csrc[pl.ds(bk * L, L)]
            pltpu.async_copy(h_hbm.at[idxv], rowbig.at[pl.ds(0, L)],
                             sem0).wait()
            _proc16(cloc[pl.ds(bk * L, L)], cw[pl.ds(bk * L, L)], 0,
                    rowbig)
            return 0

        lax.fori_loop(0, nb_f, _fblock, 0)

        def _addb(r, _):
            for k in range(D // L):
                acc[r, pl.ds(k * L, L)] = (
                    acc[r, pl.ds(k * L, L)] + biasv[pl.ds(k * L, L)])
            return 0
        lax.fori_loop(0, ROWS_PER_TEC, _addb, 0)

        for t in range(out_copies):
            loc0 = t * OUT_CHUNK
            g0 = base + loc0

            @pl.when(g0 < n_nodes)
            def _():
                pltpu.sync_copy(acc.at[pl.ds(loc0, OUT_CHUNK)],
                                out_hbm.at[pl.ds(g0, OUT_CHUNK)])

    return sc_agg


def kernel(x, edge_index, edge_weight, W, b):
    h = _matmul(x, W)
    dst = edge_index[0]
    src = edge_index[1]
    agg = _make_sc_agg(x.shape[0], src.shape[0])
    return agg(h, dst, src, edge_weight, b)


# speedup vs baseline: 1.0850x; 1.0850x over previous
import functools

import jax
import jax.numpy as jnp
from jax import lax
from jax.experimental import pallas as pl
from jax.experimental.pallas import tpu as pltpu
from jax.experimental.pallas import tpu_sc as plsc

D = 256
L = 16
NSC = 2
NTEC = 16
ROWS_PER_TEC = 320
OUT_CHUNK = 40
CHUNK = 2000
CBUF = 2176
FILT_UNROLL = 5
BG = 128


def _mm_body(x_ref, w_ref, o_ref):
    o_ref[...] = jnp.dot(x_ref[...], w_ref[...],
                         preferred_element_type=jnp.float32)


def _matmul(x, W):
    n, d = x.shape
    blk = 1000
    return pl.pallas_call(
        _mm_body,
        grid=(n // blk,),
        in_specs=[
            pl.BlockSpec((blk, d), lambda i: (i, 0)),
            pl.BlockSpec((d, W.shape[1]), lambda i: (0, 0)),
        ],
        out_specs=pl.BlockSpec((blk, W.shape[1]), lambda i: (i, 0)),
        out_shape=jax.ShapeDtypeStruct((n, W.shape[1]), jnp.float32),
    )(x, W)


def _make_sc_agg(n_nodes, n_edges):
    n_chunks = n_edges // CHUNK
    filt_iters = CHUNK // L
    out_copies = ROWS_PER_TEC // OUT_CHUNK
    mesh = plsc.VectorSubcoreMesh(core_axis_name="c", subcore_axis_name="s")

    @functools.partial(
        pl.kernel,
        out_type=jax.ShapeDtypeStruct((n_nodes, D), jnp.float32),
        mesh=mesh,
        compiler_params=pltpu.CompilerParams(needs_layout_passes=False),
        scratch_types=[
            pltpu.VMEM((ROWS_PER_TEC, D), jnp.float32),
            pltpu.VMEM((CHUNK,), jnp.int32),
            pltpu.VMEM((CHUNK,), jnp.int32),
            pltpu.VMEM((CHUNK,), jnp.float32),
            pltpu.VMEM((CBUF,), jnp.int32),
            pltpu.VMEM((CBUF,), jnp.int32),
            pltpu.VMEM((CBUF,), jnp.float32),
            pltpu.VMEM((BG, D), jnp.float32),
            pltpu.VMEM((BG,), jnp.int32),
            pltpu.VMEM((D,), jnp.float32),
            pltpu.VMEM((L,), jnp.int32),
            pltpu.SemaphoreType.DMA,
            pltpu.SemaphoreType.DMA,
            pltpu.SemaphoreType.DMA,
        ],
    )
    def sc_agg(h_hbm, dst_hbm, src_hbm, w_hbm, b_hbm, out_hbm,
               acc, dstb, srcb, wb, cloc, csrc, cw, rowbig, idxbuf,
               biasv, ctmp, sem0, sem1, esem):
        c = lax.axis_index("c")
        s = lax.axis_index("s")
        wid = c * NTEC + s
        base = wid * ROWS_PER_TEC
        zf = jnp.zeros((L,), jnp.float32)
        zi = jnp.zeros((L,), jnp.int32)
        iota = lax.iota(jnp.int32, L)

        def _zrow(r, _):
            for k in range(D // L):
                acc[r, pl.ds(k * L, L)] = zf
            return 0
        lax.fori_loop(0, ROWS_PER_TEC, _zrow, 0)
        pltpu.sync_copy(b_hbm, biasv)

        last15 = jnp.full((L,), L - 1, jnp.int32)

        def _proc16(locv, wv, row0, buf):
            for r in range(L):
                loc_s = locv[r]
                wrv = jnp.full((L,), wv[r])
                row = row0 + r

                @plsc.parallel_loop(0, D // L, step=1, unroll=16)
                def _k(k):
                    v = buf[row, pl.ds(k * L, L)] * wrv
                    plsc.addupdate(acc.at[loc_s, pl.ds(k * L, L)], v)

        def _fire_loads(ch):
            off0 = ch * CHUNK
            pltpu.async_copy(dst_hbm.at[pl.ds(off0, CHUNK)], dstb, esem)
            pltpu.async_copy(src_hbm.at[pl.ds(off0, CHUNK)], srcb, esem)
            pltpu.async_copy(w_hbm.at[pl.ds(off0, CHUNK)], wb, esem)

        _fire_loads(0)

        def _chunk(ch, cnt_in):
            off0 = ch * CHUNK
            for _ in range(3):
                pltpu.make_async_copy(dst_hbm.at[pl.ds(off0, CHUNK)],
                                      dstb, esem).wait()

            @plsc.parallel_loop(0, filt_iters, step=1, unroll=FILT_UNROLL,
                                carry=cnt_in)
            def _filt(j, off):
                d = dstb[pl.ds(j * L, L)]
                sv = srcb[pl.ds(j * L, L)]
                wv = wb[pl.ds(j * L, L)]
                loc = d - base
                m = (loc >= 0) & (loc < ROWS_PER_TEC)
                mi = jnp.where(m, 1, 0)
                cum = plsc.cumsum(mi)
                pos = off + cum - 1
                plsc.store_scatter(cloc, [pos], loc, mask=m)
                plsc.store_scatter(csrc, [pos], sv, mask=m)
                plsc.store_scatter(cw, [pos], wv, mask=m)
                return off + cum[L - 1]

            cnt = _filt

            @pl.when(ch + 1 < n_chunks)
            def _():
                _fire_loads(ch + 1)

            ndrain = cnt // BG

            def _dg(dg, _):
                b0 = dg * BG
                for k in range(BG // L):
                    idxbuf[pl.ds(k * L, L)] = csrc[pl.ds(b0 + k * L, L)]
                pltpu.async_copy(h_hbm.at[idxbuf], rowbig, sem0).wait()

                def _pblock(i, _):
                    bk16 = b0 + i * L
                    _proc16(cloc[pl.ds(bk16, L)], cw[pl.ds(bk16, L)],
                            i * L, rowbig)
                    return 0

                lax.fori_loop(0, BG // L, _pblock, 0)
                return 0

            lax.fori_loop(0, ndrain, _dg, 0)

            rem0 = ndrain * BG

            @pl.when(ndrain > 0)
            def _():
                for k in range(BG // L):
                    src_sl = pl.ds(rem0 + k * L, L)
                    dst_sl = pl.ds(k * L, L)
                    cloc[dst_sl] = cloc[src_sl]
                    csrc[dst_sl] = csrc[src_sl]
                    cw[dst_sl] = cw[src_sl]

            return cnt - rem0

        rem = lax.fori_loop(0, n_chunks, _chunk, jnp.int32(0))

        cloc[pl.ds(rem, L)] = zi
        csrc[pl.ds(rem, L)] = zi
        cw[pl.ds(rem, L)] = zf
        nb_f = (rem + (L - 1)) // L

        def _fblock(bk, _):
            idxv = csrc[pl.ds(bk * L, L)]
            pltpu.async_copy(h_hbm.at[idxv], rowbig.at[pl.ds(0, L)],
                             sem0).wait()
            _proc16(cloc[pl.ds(bk * L, L)], cw[pl.ds(bk * L, L)], 0,
                    rowbig)
            return 0

        lax.fori_loop(0, nb_f, _fblock, 0)

        def _addb(r, _):
            for k in range(D // L):
                acc[r, pl.ds(k * L, L)] = (
                    acc[r, pl.ds(k * L, L)] + biasv[pl.ds(k * L, L)])
            return 0
        lax.fori_loop(0, ROWS_PER_TEC, _addb, 0)

        for t in range(out_copies):
            loc0 = t * OUT_CHUNK
            g0 = base + loc0

            @pl.when(g0 < n_nodes)
            def _():
                pltpu.sync_copy(acc.at[pl.ds(loc0, OUT_CHUNK)],
                                out_hbm.at[pl.ds(g0, OUT_CHUNK)])

    return sc_agg


def kernel(x, edge_index, edge_weight, W, b):
    h = _matmul(x, W)
    dst = edge_index[0]
    src = edge_index[1]
    agg = _make_sc_agg(x.shape[0], src.shape[0])
    return agg(h, dst, src, edge_weight, b)


# speedup vs baseline: 1.2784x; 1.1783x over previous
import functools

import jax
import jax.numpy as jnp
from jax import lax
from jax.experimental import pallas as pl
from jax.experimental.pallas import tpu as pltpu
from jax.experimental.pallas import tpu_sc as plsc

D = 256
L = 16
NSC = 2
NTEC = 16
ROWS_PER_TEC = 320
OUT_CHUNK = 40
CHUNK = 2000
CBUF = 2176
FILT_UNROLL = 5
BG = 128


def _mm_body(x_ref, w_ref, o_ref):
    o_ref[...] = jnp.dot(x_ref[...], w_ref[...],
                         preferred_element_type=jnp.float32)


def _matmul(x, W):
    n, d = x.shape
    blk = 1000
    return pl.pallas_call(
        _mm_body,
        grid=(n // blk,),
        in_specs=[
            pl.BlockSpec((blk, d), lambda i: (i, 0)),
            pl.BlockSpec((d, W.shape[1]), lambda i: (0, 0)),
        ],
        out_specs=pl.BlockSpec((blk, W.shape[1]), lambda i: (i, 0)),
        out_shape=jax.ShapeDtypeStruct((n, W.shape[1]), jnp.float32),
    )(x, W)


def _make_sc_agg(n_nodes, n_edges):
    n_chunks = n_edges // CHUNK
    filt_iters = CHUNK // L
    out_copies = ROWS_PER_TEC // OUT_CHUNK
    mesh = plsc.VectorSubcoreMesh(core_axis_name="c", subcore_axis_name="s")

    @functools.partial(
        pl.kernel,
        out_type=jax.ShapeDtypeStruct((n_nodes, D), jnp.float32),
        mesh=mesh,
        compiler_params=pltpu.CompilerParams(needs_layout_passes=False),
        scratch_types=[
            pltpu.VMEM((ROWS_PER_TEC, D), jnp.float32),
            pltpu.VMEM((CHUNK,), jnp.int32),
            pltpu.VMEM((CHUNK,), jnp.int32),
            pltpu.VMEM((CHUNK,), jnp.float32),
            pltpu.VMEM((CBUF,), jnp.int32),
            pltpu.VMEM((CBUF,), jnp.int32),
            pltpu.VMEM((CBUF,), jnp.float32),
            pltpu.VMEM((BG, D), jnp.float32),
            pltpu.VMEM((BG,), jnp.int32),
            pltpu.VMEM((BG,), jnp.int32),
            pltpu.VMEM((BG,), jnp.float32),
            pltpu.VMEM((D,), jnp.float32),
            pltpu.VMEM((L,), jnp.int32),
            pltpu.SemaphoreType.DMA,
            pltpu.SemaphoreType.DMA,
            pltpu.SemaphoreType.DMA,
        ],
    )
    def sc_agg(h_hbm, dst_hbm, src_hbm, w_hbm, b_hbm, out_hbm,
               acc, dstb, srcb, wb, cloc, csrc, cw, rowbig, idxbuf,
               pend_loc, pend_w, biasv, ctmp, sem0, sem1, esem):
        c = lax.axis_index("c")
        s = lax.axis_index("s")
        wid = c * NTEC + s
        base = wid * ROWS_PER_TEC
        zf = jnp.zeros((L,), jnp.float32)
        zi = jnp.zeros((L,), jnp.int32)
        iota = lax.iota(jnp.int32, L)

        def _zrow(r, _):
            for k in range(D // L):
                acc[r, pl.ds(k * L, L)] = zf
            return 0
        lax.fori_loop(0, ROWS_PER_TEC, _zrow, 0)
        pltpu.sync_copy(b_hbm, biasv)

        last15 = jnp.full((L,), L - 1, jnp.int32)

        def _proc16(locv, wv, row0, buf):
            for r in range(L):
                loc_s = locv[r]
                wrv = jnp.full((L,), wv[r])
                row = row0 + r

                @plsc.parallel_loop(0, D // L, step=1, unroll=16)
                def _k(k):
                    v = buf[row, pl.ds(k * L, L)] * wrv
                    plsc.addupdate(acc.at[loc_s, pl.ds(k * L, L)], v)

        def _fire_loads(ch):
            off0 = ch * CHUNK
            pltpu.async_copy(dst_hbm.at[pl.ds(off0, CHUNK)], dstb, esem)
            pltpu.async_copy(src_hbm.at[pl.ds(off0, CHUNK)], srcb, esem)
            pltpu.async_copy(w_hbm.at[pl.ds(off0, CHUNK)], wb, esem)

        _fire_loads(0)

        def _wait_and_process():
            pltpu.make_async_copy(h_hbm.at[pl.ds(0, BG)], rowbig,
                                  sem0).wait()

            def _pblock(i, _):
                _proc16(pend_loc[pl.ds(i * L, L)],
                        pend_w[pl.ds(i * L, L)], i * L, rowbig)
                return 0

            lax.fori_loop(0, BG // L, _pblock, 0)

        def _chunk(ch, carry):
            cnt_in, pend_in = carry
            off0 = ch * CHUNK
            for _ in range(3):
                pltpu.make_async_copy(dst_hbm.at[pl.ds(off0, CHUNK)],
                                      dstb, esem).wait()

            @plsc.parallel_loop(0, filt_iters, step=1, unroll=FILT_UNROLL,
                                carry=cnt_in)
            def _filt(j, off):
                d = dstb[pl.ds(j * L, L)]
                sv = srcb[pl.ds(j * L, L)]
                wv = wb[pl.ds(j * L, L)]
                loc = d - base
                m = (loc >= 0) & (loc < ROWS_PER_TEC)
                mi = jnp.where(m, 1, 0)
                cum = plsc.cumsum(mi)
                pos = off + cum - 1
                plsc.store_scatter(cloc, [pos], loc, mask=m)
                plsc.store_scatter(csrc, [pos], sv, mask=m)
                plsc.store_scatter(cw, [pos], wv, mask=m)
                return off + cum[L - 1]

            cnt = _filt

            @pl.when(ch + 1 < n_chunks)
            def _():
                _fire_loads(ch + 1)

            ndrain = cnt // BG

            def _dg(dg, p):
                @pl.when(p > 0)
                def _():
                    _wait_and_process()

                b0 = dg * BG
                for k in range(BG // L):
                    idxbuf[pl.ds(k * L, L)] = csrc[pl.ds(b0 + k * L, L)]
                    pend_loc[pl.ds(k * L, L)] = cloc[pl.ds(b0 + k * L, L)]
                    pend_w[pl.ds(k * L, L)] = cw[pl.ds(b0 + k * L, L)]
                pltpu.async_copy(h_hbm.at[idxbuf], rowbig, sem0)
                return jnp.int32(1)

            pend_out = lax.fori_loop(0, ndrain, _dg, pend_in)

            rem0 = ndrain * BG

            @pl.when(ndrain > 0)
            def _():
                for k in range(BG // L):
                    src_sl = pl.ds(rem0 + k * L, L)
                    dst_sl = pl.ds(k * L, L)
                    cloc[dst_sl] = cloc[src_sl]
                    csrc[dst_sl] = csrc[src_sl]
                    cw[dst_sl] = cw[src_sl]

            return cnt - rem0, pend_out

        rem, pend = lax.fori_loop(0, n_chunks, _chunk,
                                  (jnp.int32(0), jnp.int32(0)))

        @pl.when(pend > 0)
        def _():
            _wait_and_process()

        cloc[pl.ds(rem, L)] = zi
        csrc[pl.ds(rem, L)] = zi
        cw[pl.ds(rem, L)] = zf
        nb_f = (rem + (L - 1)) // L

        def _fblock(bk, _):
            idxv = csrc[pl.ds(bk * L, L)]
            pltpu.async_copy(h_hbm.at[idxv], rowbig.at[pl.ds(0, L)],
                             sem0).wait()
            _proc16(cloc[pl.ds(bk * L, L)], cw[pl.ds(bk * L, L)], 0,
                    rowbig)
            return 0

        lax.fori_loop(0, nb_f, _fblock, 0)

        def _addb(r, _):
            for k in range(D // L):
                acc[r, pl.ds(k * L, L)] = (
                    acc[r, pl.ds(k * L, L)] + biasv[pl.ds(k * L, L)])
            return 0
        lax.fori_loop(0, ROWS_PER_TEC, _addb, 0)

        for t in range(out_copies):
            loc0 = t * OUT_CHUNK
            g0 = base + loc0

            @pl.when(g0 < n_nodes)
            def _():
                pltpu.sync_copy(acc.at[pl.ds(loc0, OUT_CHUNK)],
                                out_hbm.at[pl.ds(g0, OUT_CHUNK)])

    return sc_agg


def kernel(x, edge_index, edge_weight, W, b):
    h = _matmul(x, W)
    dst = edge_index[0]
    src = edge_index[1]
    agg = _make_sc_agg(x.shape[0], src.shape[0])
    return agg(h, dst, src, edge_weight, b)


# speedup vs baseline: 1.2795x; 1.0009x over previous
import functools

import jax
import jax.numpy as jnp
from jax import lax
from jax.experimental import pallas as pl
from jax.experimental.pallas import tpu as pltpu
from jax.experimental.pallas import tpu_sc as plsc

D = 256
L = 16
NSC = 2
NTEC = 16
ROWS_PER_TEC = 320
OUT_CHUNK = 40
CHUNK = 2000
CBUF = 2176
FILT_UNROLL = 5
BG = 128


def _mm_body(x_ref, w_ref, o_ref):
    o_ref[...] = jnp.dot(x_ref[...], w_ref[...],
                         preferred_element_type=jnp.float32)


def _matmul(x, W):
    n, d = x.shape
    blk = 1000
    return pl.pallas_call(
        _mm_body,
        grid=(n // blk,),
        in_specs=[
            pl.BlockSpec((blk, d), lambda i: (i, 0)),
            pl.BlockSpec((d, W.shape[1]), lambda i: (0, 0)),
        ],
        out_specs=pl.BlockSpec((blk, W.shape[1]), lambda i: (i, 0)),
        out_shape=jax.ShapeDtypeStruct((n, W.shape[1]), jnp.float32),
    )(x, W)


def _make_sc_agg(n_nodes, n_edges):
    n_chunks = n_edges // CHUNK
    filt_iters = CHUNK // L
    out_copies = ROWS_PER_TEC // OUT_CHUNK
    mesh = plsc.VectorSubcoreMesh(core_axis_name="c", subcore_axis_name="s")

    @functools.partial(
        pl.kernel,
        out_type=jax.ShapeDtypeStruct((n_nodes, D), jnp.float32),
        mesh=mesh,
        compiler_params=pltpu.CompilerParams(needs_layout_passes=False),
        scratch_types=[
            pltpu.VMEM((ROWS_PER_TEC, D), jnp.float32),
            pltpu.VMEM((CHUNK,), jnp.int32),
            pltpu.VMEM((CHUNK,), jnp.int32),
            pltpu.VMEM((CHUNK,), jnp.float32),
            pltpu.VMEM((CBUF,), jnp.int32),
            pltpu.VMEM((CBUF,), jnp.int32),
            pltpu.VMEM((CBUF,), jnp.float32),
            pltpu.VMEM((BG, D), jnp.float32),
            pltpu.VMEM((BG,), jnp.int32),
            pltpu.VMEM((BG,), jnp.int32),
            pltpu.VMEM((BG,), jnp.float32),
            pltpu.VMEM((D,), jnp.float32),
            pltpu.VMEM((L,), jnp.int32),
            pltpu.SemaphoreType.DMA,
            pltpu.SemaphoreType.DMA,
            pltpu.SemaphoreType.DMA,
        ],
    )
    def sc_agg(h_hbm, dst_hbm, src_hbm, w_hbm, b_hbm, out_hbm,
               acc, dstb, srcb, wb, cloc, csrc, cw, rowbig, idxbuf,
               pend_loc, pend_w, biasv, ctmp, sem0, sem1, esem):
        c = lax.axis_index("c")
        s = lax.axis_index("s")
        wid = c * NTEC + s
        base = wid * ROWS_PER_TEC
        zf = jnp.zeros((L,), jnp.float32)
        zi = jnp.zeros((L,), jnp.int32)
        iota = lax.iota(jnp.int32, L)

        def _zrow(r, _):
            for k in range(D // L):
                acc[r, pl.ds(k * L, L)] = zf
            return 0
        lax.fori_loop(0, ROWS_PER_TEC, _zrow, 0)
        pltpu.sync_copy(b_hbm, biasv)

        last15 = jnp.full((L,), L - 1, jnp.int32)

        def _proc16(locv, wv, row0, buf):
            for r in range(L):
                loc_s = locv[r]
                wrv = jnp.full((L,), wv[r])
                row = row0 + r

                @plsc.parallel_loop(0, D // L, step=1, unroll=16)
                def _k(k):
                    v = buf[row, pl.ds(k * L, L)] * wrv
                    plsc.addupdate(acc.at[loc_s, pl.ds(k * L, L)], v)

        def _fire_loads(ch):
            off0 = ch * CHUNK
            pltpu.async_copy(dst_hbm.at[pl.ds(off0, CHUNK)], dstb, esem)
            pltpu.async_copy(src_hbm.at[pl.ds(off0, CHUNK)], srcb, esem)
            pltpu.async_copy(w_hbm.at[pl.ds(off0, CHUNK)], wb, esem)

        _fire_loads(0)

        def _wait_and_process():
            pltpu.make_async_copy(h_hbm.at[pl.ds(0, BG)], rowbig,
                                  sem0).wait()

            def _pblock(i, _):
                _proc16(pend_loc[pl.ds(i * L, L)],
                        pend_w[pl.ds(i * L, L)], i * L, rowbig)
                return 0

            lax.fori_loop(0, BG // L, _pblock, 0)

        def _chunk(ch, carry):
            cnt_in, pend_in = carry
            off0 = ch * CHUNK
            for _ in range(3):
                pltpu.make_async_copy(dst_hbm.at[pl.ds(off0, CHUNK)],
                                      dstb, esem).wait()

            @plsc.parallel_loop(0, filt_iters, step=1, unroll=FILT_UNROLL,
                                carry=cnt_in)
            def _filt(j, off):
                d = dstb[pl.ds(j * L, L)]
                sv = srcb[pl.ds(j * L, L)]
                wv = wb[pl.ds(j * L, L)]
                loc = d - base
                m = (loc >= 0) & (loc < ROWS_PER_TEC)
                plsc.store_compressed(cloc.at[pl.ds(off, L)], loc, mask=m)
                plsc.store_compressed(csrc.at[pl.ds(off, L)], sv, mask=m)
                plsc.store_compressed(cw.at[pl.ds(off, L)], wv, mask=m)
                return off + plsc.all_reduce_population_count(m)[0]

            cnt = _filt

            @pl.when(ch + 1 < n_chunks)
            def _():
                _fire_loads(ch + 1)

            ndrain = cnt // BG

            def _dg(dg, p):
                @pl.when(p > 0)
                def _():
                    _wait_and_process()

                b0 = dg * BG
                for k in range(BG // L):
                    idxbuf[pl.ds(k * L, L)] = csrc[pl.ds(b0 + k * L, L)]
                    pend_loc[pl.ds(k * L, L)] = cloc[pl.ds(b0 + k * L, L)]
                    pend_w[pl.ds(k * L, L)] = cw[pl.ds(b0 + k * L, L)]
                pltpu.async_copy(h_hbm.at[idxbuf], rowbig, sem0)
                return jnp.int32(1)

            pend_out = lax.fori_loop(0, ndrain, _dg, pend_in)

            rem0 = ndrain * BG

            @pl.when(ndrain > 0)
            def _():
                for k in range(BG // L):
                    src_sl = pl.ds(rem0 + k * L, L)
                    dst_sl = pl.ds(k * L, L)
                    cloc[dst_sl] = cloc[src_sl]
                    csrc[dst_sl] = csrc[src_sl]
                    cw[dst_sl] = cw[src_sl]

            return cnt - rem0, pend_out

        rem, pend = lax.fori_loop(0, n_chunks, _chunk,
                                  (jnp.int32(0), jnp.int32(0)))

        @pl.when(pend > 0)
        def _():
            _wait_and_process()

        cloc[pl.ds(rem, L)] = zi
        csrc[pl.ds(rem, L)] = zi
        cw[pl.ds(rem, L)] = zf
        nb_f = (rem + (L - 1)) // L

        def _fblock(bk, _):
            idxv = csrc[pl.ds(bk * L, L)]
            pltpu.async_copy(h_hbm.at[idxv], rowbig.at[pl.ds(0, L)],
                             sem0).wait()
            _proc16(cloc[pl.ds(bk * L, L)], cw[pl.ds(bk * L, L)], 0,
                    rowbig)
            return 0

        lax.fori_loop(0, nb_f, _fblock, 0)

        def _addb(r, _):
            for k in range(D // L):
                acc[r, pl.ds(k * L, L)] = (
                    acc[r, pl.ds(k * L, L)] + biasv[pl.ds(k * L, L)])
            return 0
        lax.fori_loop(0, ROWS_PER_TEC, _addb, 0)

        for t in range(out_copies):
            loc0 = t * OUT_CHUNK
            g0 = base + loc0

            @pl.when(g0 < n_nodes)
            def _():
                pltpu.sync_copy(acc.at[pl.ds(loc0, OUT_CHUNK)],
                                out_hbm.at[pl.ds(g0, OUT_CHUNK)])

    return sc_agg


def kernel(x, edge_index, edge_weight, W, b):
    h = _matmul(x, W)
    dst = edge_index[0]
    src = edge_index[1]
    agg = _make_sc_agg(x.shape[0], src.shape[0])
    return agg(h, dst, src, edge_weight, b)


# speedup vs baseline: 1.3515x; 1.0563x over previous
import functools

import jax
import jax.numpy as jnp
from jax import lax
from jax.experimental import pallas as pl
from jax.experimental.pallas import tpu as pltpu
from jax.experimental.pallas import tpu_sc as plsc

D = 256
L = 16
NSC = 2
NTEC = 16
ROWS_PER_TEC = 320
OUT_CHUNK = 40
CHUNK = 2000
CBUF = 2176
FILT_UNROLL = 5
BG = 128


def _mm_body(x_ref, w_ref, o_ref):
    o_ref[...] = jnp.dot(x_ref[...], w_ref[...],
                         preferred_element_type=jnp.float32)


def _matmul(x, W):
    n, d = x.shape
    blk = 1000
    return pl.pallas_call(
        _mm_body,
        grid=(n // blk,),
        in_specs=[
            pl.BlockSpec((blk, d), lambda i: (i, 0)),
            pl.BlockSpec((d, W.shape[1]), lambda i: (0, 0)),
        ],
        out_specs=pl.BlockSpec((blk, W.shape[1]), lambda i: (i, 0)),
        out_shape=jax.ShapeDtypeStruct((n, W.shape[1]), jnp.float32),
    )(x, W)


def _make_sc_agg(n_nodes, n_edges):
    n_chunks = n_edges // CHUNK
    filt_iters = CHUNK // L
    out_copies = ROWS_PER_TEC // OUT_CHUNK
    mesh = plsc.VectorSubcoreMesh(core_axis_name="c", subcore_axis_name="s")

    @functools.partial(
        pl.kernel,
        out_type=jax.ShapeDtypeStruct((n_nodes, D), jnp.float32),
        mesh=mesh,
        compiler_params=pltpu.CompilerParams(needs_layout_passes=False),
        scratch_types=[
            pltpu.VMEM((ROWS_PER_TEC, D), jnp.float32),
            pltpu.VMEM((CHUNK,), jnp.int32),
            pltpu.VMEM((CHUNK,), jnp.int32),
            pltpu.VMEM((CHUNK,), jnp.float32),
            pltpu.VMEM((CBUF,), jnp.int32),
            pltpu.VMEM((CBUF,), jnp.int32),
            pltpu.VMEM((CBUF,), jnp.float32),
            pltpu.VMEM((BG, D), jnp.float32),
            pltpu.VMEM((BG,), jnp.int32),
            pltpu.VMEM((BG,), jnp.int32),
            pltpu.VMEM((BG,), jnp.float32),
            pltpu.VMEM((D,), jnp.float32),
            pltpu.VMEM((L,), jnp.int32),
            pltpu.SemaphoreType.DMA,
            pltpu.SemaphoreType.DMA,
            pltpu.SemaphoreType.DMA,
        ],
    )
    def sc_agg(h_hbm, dst_hbm, src_hbm, w_hbm, b_hbm, out_hbm,
               acc, dstb, srcb, wb, cloc, csrc, cw, rowbig, idxbuf,
               pend_loc, pend_w, biasv, ctmp, sem0, sem1, esem):
        c = lax.axis_index("c")
        s = lax.axis_index("s")
        wid = c * NTEC + s
        base = wid * ROWS_PER_TEC
        zf = jnp.zeros((L,), jnp.float32)
        zi = jnp.zeros((L,), jnp.int32)
        iota = lax.iota(jnp.int32, L)

        @plsc.parallel_loop(0, ROWS_PER_TEC, step=1, unroll=4)
        def _zrow(r):
            for k in range(D // L):
                acc[r, pl.ds(k * L, L)] = zf
        pltpu.sync_copy(b_hbm, biasv)

        last15 = jnp.full((L,), L - 1, jnp.int32)

        def _proc16(locv, wv, row0, buf):
            for r in range(L):
                loc_s = locv[r]
                wrv = jnp.full((L,), wv[r])
                row = row0 + r

                @plsc.parallel_loop(0, D // L, step=1, unroll=16)
                def _k(k):
                    v = buf[row, pl.ds(k * L, L)] * wrv
                    plsc.addupdate(acc.at[loc_s, pl.ds(k * L, L)], v)

        def _fire_loads(ch):
            off0 = ch * CHUNK
            pltpu.async_copy(dst_hbm.at[pl.ds(off0, CHUNK)], dstb, esem)
            pltpu.async_copy(src_hbm.at[pl.ds(off0, CHUNK)], srcb, esem)
            pltpu.async_copy(w_hbm.at[pl.ds(off0, CHUNK)], wb, esem)

        _fire_loads(0)

        def _wait_and_process():
            pltpu.make_async_copy(h_hbm.at[pl.ds(0, BG)], rowbig,
                                  sem0).wait()

            def _pblock(i, _):
                _proc16(pend_loc[pl.ds(i * L, L)],
                        pend_w[pl.ds(i * L, L)], i * L, rowbig)
                return 0

            lax.fori_loop(0, BG // L, _pblock, 0)

        def _chunk(ch, carry):
            cnt_in, pend_in = carry
            off0 = ch * CHUNK
            for _ in range(3):
                pltpu.make_async_copy(dst_hbm.at[pl.ds(off0, CHUNK)],
                                      dstb, esem).wait()

            @plsc.parallel_loop(0, filt_iters, step=1, unroll=FILT_UNROLL,
                                carry=cnt_in)
            def _filt(j, off):
                d = dstb[pl.ds(j * L, L)]
                sv = srcb[pl.ds(j * L, L)]
                wv = wb[pl.ds(j * L, L)]
                loc = d - base
                m = (loc >= 0) & (loc < ROWS_PER_TEC)
                plsc.store_compressed(cloc.at[pl.ds(off, L)], loc, mask=m)
                plsc.store_compressed(csrc.at[pl.ds(off, L)], sv, mask=m)
                plsc.store_compressed(cw.at[pl.ds(off, L)], wv, mask=m)
                return off + plsc.all_reduce_population_count(m)[0]

            cnt = _filt

            @pl.when(ch + 1 < n_chunks)
            def _():
                _fire_loads(ch + 1)

            ndrain = cnt // BG

            def _dg(dg, p):
                @pl.when(p > 0)
                def _():
                    _wait_and_process()

                b0 = dg * BG
                for k in range(BG // L):
                    idxbuf[pl.ds(k * L, L)] = csrc[pl.ds(b0 + k * L, L)]
                    pend_loc[pl.ds(k * L, L)] = cloc[pl.ds(b0 + k * L, L)]
                    pend_w[pl.ds(k * L, L)] = cw[pl.ds(b0 + k * L, L)]
                pltpu.async_copy(h_hbm.at[idxbuf], rowbig, sem0)
                return jnp.int32(1)

            pend_out = lax.fori_loop(0, ndrain, _dg, pend_in)

            rem0 = ndrain * BG

            @pl.when(ndrain > 0)
            def _():
                for k in range(BG // L):
                    src_sl = pl.ds(rem0 + k * L, L)
                    dst_sl = pl.ds(k * L, L)
                    cloc[dst_sl] = cloc[src_sl]
                    csrc[dst_sl] = csrc[src_sl]
                    cw[dst_sl] = cw[src_sl]

            return cnt - rem0, pend_out

        rem, pend = lax.fori_loop(0, n_chunks, _chunk,
                                  (jnp.int32(0), jnp.int32(0)))

        @pl.when(pend > 0)
        def _():
            _wait_and_process()

        cloc[pl.ds(rem, L)] = zi
        csrc[pl.ds(rem, L)] = zi
        cw[pl.ds(rem, L)] = zf
        nb_f = (rem + (L - 1)) // L

        def _fblock(bk, _):
            idxv = csrc[pl.ds(bk * L, L)]
            pltpu.async_copy(h_hbm.at[idxv], rowbig.at[pl.ds(0, L)],
                             sem0).wait()
            _proc16(cloc[pl.ds(bk * L, L)], cw[pl.ds(bk * L, L)], 0,
                    rowbig)
            return 0

        lax.fori_loop(0, nb_f, _fblock, 0)

        @plsc.parallel_loop(0, ROWS_PER_TEC, step=1, unroll=4)
        def _addb(r):
            for k in range(D // L):
                acc[r, pl.ds(k * L, L)] = (
                    acc[r, pl.ds(k * L, L)] + biasv[pl.ds(k * L, L)])

        for t in range(out_copies):
            loc0 = t * OUT_CHUNK
            g0 = base + loc0

            @pl.when(g0 < n_nodes)
            def _():
                pltpu.async_copy(acc.at[pl.ds(loc0, OUT_CHUNK)],
                                 out_hbm.at[pl.ds(g0, OUT_CHUNK)], sem1)

        for t in range(out_copies):
            loc0 = t * OUT_CHUNK
            g0 = base + loc0

            @pl.when(g0 < n_nodes)
            def _():
                pltpu.make_async_copy(
                    acc.at[pl.ds(loc0, OUT_CHUNK)],
                    out_hbm.at[pl.ds(g0, OUT_CHUNK)], sem1).wait()

    return sc_agg


def kernel(x, edge_index, edge_weight, W, b):
    h = _matmul(x, W)
    dst = edge_index[0]
    src = edge_index[1]
    agg = _make_sc_agg(x.shape[0], src.shape[0])
    return agg(h, dst, src, edge_weight, b)


# speedup vs baseline: 1.3648x; 1.0098x over previous
import functools

import jax
import jax.numpy as jnp
from jax import lax
from jax.experimental import pallas as pl
from jax.experimental.pallas import tpu as pltpu
from jax.experimental.pallas import tpu_sc as plsc

D = 256
L = 16
NSC = 2
NTEC = 16
ROWS_PER_TEC = 320
OUT_CHUNK = 40
CHUNK = 2000
CBUF = 2176
FILT_UNROLL = 5
BG = 128


def _mm_body(x_ref, w_ref, o_ref):
    o_ref[...] = jnp.dot(x_ref[...], w_ref[...],
                         preferred_element_type=jnp.float32)


def _matmul(x, W):
    n, d = x.shape
    blk = 2000
    return pl.pallas_call(
        _mm_body,
        grid=(n // blk,),
        in_specs=[
            pl.BlockSpec((blk, d), lambda i: (i, 0)),
            pl.BlockSpec((d, W.shape[1]), lambda i: (0, 0)),
        ],
        out_specs=pl.BlockSpec((blk, W.shape[1]), lambda i: (i, 0)),
        out_shape=jax.ShapeDtypeStruct((n, W.shape[1]), jnp.float32),
    )(x, W)


def _make_sc_agg(n_nodes, n_edges):
    n_chunks = n_edges // CHUNK
    filt_iters = CHUNK // L
    out_copies = ROWS_PER_TEC // OUT_CHUNK
    mesh = plsc.VectorSubcoreMesh(core_axis_name="c", subcore_axis_name="s")

    @functools.partial(
        pl.kernel,
        out_type=jax.ShapeDtypeStruct((n_nodes, D), jnp.float32),
        mesh=mesh,
        compiler_params=pltpu.CompilerParams(needs_layout_passes=False),
        scratch_types=[
            pltpu.VMEM((ROWS_PER_TEC, D), jnp.float32),
            pltpu.VMEM((CHUNK,), jnp.int32),
            pltpu.VMEM((CHUNK,), jnp.int32),
            pltpu.VMEM((CHUNK,), jnp.float32),
            pltpu.VMEM((CBUF,), jnp.int32),
            pltpu.VMEM((CBUF,), jnp.int32),
            pltpu.VMEM((CBUF,), jnp.float32),
            pltpu.VMEM((BG, D), jnp.float32),
            pltpu.VMEM((BG,), jnp.int32),
            pltpu.VMEM((BG,), jnp.int32),
            pltpu.VMEM((BG,), jnp.float32),
            pltpu.VMEM((D,), jnp.float32),
            pltpu.VMEM((L,), jnp.int32),
            pltpu.SemaphoreType.DMA,
            pltpu.SemaphoreType.DMA,
            pltpu.SemaphoreType.DMA,
        ],
    )
    def sc_agg(h_hbm, dst_hbm, src_hbm, w_hbm, b_hbm, out_hbm,
               acc, dstb, srcb, wb, cloc, csrc, cw, rowbig, idxbuf,
               pend_loc, pend_w, biasv, ctmp, sem0, sem1, esem):
        c = lax.axis_index("c")
        s = lax.axis_index("s")
        wid = c * NTEC + s
        base = wid * ROWS_PER_TEC
        zf = jnp.zeros((L,), jnp.float32)
        zi = jnp.zeros((L,), jnp.int32)
        iota = lax.iota(jnp.int32, L)

        @plsc.parallel_loop(0, ROWS_PER_TEC, step=1, unroll=4)
        def _zrow(r):
            for k in range(D // L):
                acc[r, pl.ds(k * L, L)] = zf
        pltpu.sync_copy(b_hbm, biasv)

        last15 = jnp.full((L,), L - 1, jnp.int32)

        def _proc16(locv, wv, row0, buf):
            for r in range(L):
                loc_s = locv[r]
                wrv = jnp.full((L,), wv[r])
                row = row0 + r

                @plsc.parallel_loop(0, D // L, step=1, unroll=16)
                def _k(k):
                    v = buf[row, pl.ds(k * L, L)] * wrv
                    plsc.addupdate(acc.at[loc_s, pl.ds(k * L, L)], v)

        def _fire_loads(ch):
            off0 = ch * CHUNK
            pltpu.async_copy(dst_hbm.at[pl.ds(off0, CHUNK)], dstb, esem)
            pltpu.async_copy(src_hbm.at[pl.ds(off0, CHUNK)], srcb, esem)
            pltpu.async_copy(w_hbm.at[pl.ds(off0, CHUNK)], wb, esem)

        _fire_loads(0)

        def _wait_and_process():
            pltpu.make_async_copy(h_hbm.at[pl.ds(0, BG)], rowbig,
                                  sem0).wait()

            def _pblock(i, _):
                _proc16(pend_loc[pl.ds(i * L, L)],
                        pend_w[pl.ds(i * L, L)], i * L, rowbig)
                return 0

            lax.fori_loop(0, BG // L, _pblock, 0)

        def _chunk(ch, carry):
            cnt_in, pend_in = carry
            off0 = ch * CHUNK
            for _ in range(3):
                pltpu.make_async_copy(dst_hbm.at[pl.ds(off0, CHUNK)],
                                      dstb, esem).wait()

            @plsc.parallel_loop(0, filt_iters, step=1, unroll=FILT_UNROLL,
                                carry=cnt_in)
            def _filt(j, off):
                d = dstb[pl.ds(j * L, L)]
                sv = srcb[pl.ds(j * L, L)]
                wv = wb[pl.ds(j * L, L)]
                loc = d - base
                m = (loc >= 0) & (loc < ROWS_PER_TEC)
                plsc.store_compressed(cloc.at[pl.ds(off, L)], loc, mask=m)
                plsc.store_compressed(csrc.at[pl.ds(off, L)], sv, mask=m)
                plsc.store_compressed(cw.at[pl.ds(off, L)], wv, mask=m)
                return off + plsc.all_reduce_population_count(m)[0]

            cnt = _filt

            @pl.when(ch + 1 < n_chunks)
            def _():
                _fire_loads(ch + 1)

            ndrain = cnt // BG

            def _dg(dg, p):
                @pl.when(p > 0)
                def _():
                    _wait_and_process()

                b0 = dg * BG
                for k in range(BG // L):
                    idxbuf[pl.ds(k * L, L)] = csrc[pl.ds(b0 + k * L, L)]
                    pend_loc[pl.ds(k * L, L)] = cloc[pl.ds(b0 + k * L, L)]
                    pend_w[pl.ds(k * L, L)] = cw[pl.ds(b0 + k * L, L)]
                pltpu.async_copy(h_hbm.at[idxbuf], rowbig, sem0)
                return jnp.int32(1)

            pend_out = lax.fori_loop(0, ndrain, _dg, pend_in)

            rem0 = ndrain * BG

            @pl.when(ndrain > 0)
            def _():
                for k in range(BG // L):
                    src_sl = pl.ds(rem0 + k * L, L)
                    dst_sl = pl.ds(k * L, L)
                    cloc[dst_sl] = cloc[src_sl]
                    csrc[dst_sl] = csrc[src_sl]
                    cw[dst_sl] = cw[src_sl]

            return cnt - rem0, pend_out

        rem, pend = lax.fori_loop(0, n_chunks, _chunk,
                                  (jnp.int32(0), jnp.int32(0)))

        @pl.when(pend > 0)
        def _():
            _wait_and_process()

        cloc[pl.ds(rem, L)] = zi
        csrc[pl.ds(rem, L)] = zi
        cw[pl.ds(rem, L)] = zf
        nb_f = (rem + (L - 1)) // L

        def _fblock(bk, _):
            idxv = csrc[pl.ds(bk * L, L)]
            pltpu.async_copy(h_hbm.at[idxv], rowbig.at[pl.ds(0, L)],
                             sem0).wait()
            _proc16(cloc[pl.ds(bk * L, L)], cw[pl.ds(bk * L, L)], 0,
                    rowbig)
            return 0

        lax.fori_loop(0, nb_f, _fblock, 0)

        @plsc.parallel_loop(0, ROWS_PER_TEC, step=1, unroll=4)
        def _addb(r):
            for k in range(D // L):
                acc[r, pl.ds(k * L, L)] = (
                    acc[r, pl.ds(k * L, L)] + biasv[pl.ds(k * L, L)])

        for t in range(out_copies):
            loc0 = t * OUT_CHUNK
            g0 = base + loc0

            @pl.when(g0 < n_nodes)
            def _():
                pltpu.async_copy(acc.at[pl.ds(loc0, OUT_CHUNK)],
                                 out_hbm.at[pl.ds(g0, OUT_CHUNK)], sem1)

        for t in range(out_copies):
            loc0 = t * OUT_CHUNK
            g0 = base + loc0

            @pl.when(g0 < n_nodes)
            def _():
                pltpu.make_async_copy(
                    acc.at[pl.ds(loc0, OUT_CHUNK)],
                    out_hbm.at[pl.ds(g0, OUT_CHUNK)], sem1).wait()

    return sc_agg


def kernel(x, edge_index, edge_weight, W, b):
    h = _matmul(x, W)
    dst = edge_index[0]
    src = edge_index[1]
    agg = _make_sc_agg(x.shape[0], src.shape[0])
    return agg(h, dst, src, edge_weight, b)
